# baseline (speedup 1.0000x reference)
import functools

import jax
import jax.numpy as jnp
from jax import lax
from jax.experimental import pallas as pl
from jax.experimental.pallas import tpu as pltpu
from jax.experimental.pallas import tpu_sc as plsc

_E = 64
_Q = 1.2
_ENT_TH = 2.5
_TOP_P = 0.75

_NC, _NS, _L = 2, 16, 16
_NW = _NC * _NS
_N = 65536
_TPW = _N // _NW
_CH = 512
_NCH = _TPW // _CH
_NG = _CH // _L
_PR = 2 * _E + 4


def _sc_body(x_hbm, w_hbm, out_hbm, xb, wb, pbuf, acc):
    wid = lax.axis_index("s") * _NC + lax.axis_index("c")
    zeros = jnp.zeros((_L,), jnp.float32)

    def _zinit(j, carry):
        acc[j, :] = zeros
        return carry

    lax.fori_loop(0, _PR, _zinit, 0)

    S, T, D = zeros, zeros, zeros
    for c in range(_NCH):
        pltpu.sync_copy(x_hbm.at[wid, c], xb)
        pltpu.sync_copy(w_hbm.at[wid, c], wb)

        def _group(g, std):
            S, T, D = std
            sl = pl.ds(g * _L, _L)

            def _mx(j, m):
                return jnp.maximum(m, xb[j, sl])

            m = lax.fori_loop(0, _E, _mx, jnp.full((_L,), -1e30, jnp.float32))

            def _exp(j, z):
                e = jnp.exp(xb[j, sl] - m)
                pbuf[j, sl] = e
                return z + e

            z = lax.fori_loop(0, _E, _exp, zeros)
            rz = 1.0 / z

            y0 = jnp.where(z >= 20.0855, 3.5,
                           jnp.where(z >= 7.3891, 2.5,
                                     jnp.where(z >= 2.7183, 1.5, 0.5)))

            def _newton(_, y):
                return y - 1.0 + z * jnp.exp(-y)

            lnz = lax.fori_loop(0, 5, _newton, y0)

            def _scale(j, carry):
                S, T, m1, pqs = carry
                pv = pbuf[j, sl] * rz
                pbuf[j, sl] = pv
                pc = jnp.maximum(pv, 1e-5)
                pq = jnp.maximum(jnp.exp(_Q * (xb[j, sl] - m - lnz)), 1e-6)
                return (S + pc, T + pq, jnp.maximum(m1, pv), pqs + pq)

            S, T, m1, pqs = lax.fori_loop(
                0, _E, _scale,
                (S, T, jnp.full((_L,), -1.0, jnp.float32), zeros))

            high = ((1.0 - pqs) / (_Q - 1.0)) >= _ENT_TH

            def _m2(j, m2):
                pv = pbuf[j, sl]
                return jnp.maximum(m2, jnp.where(pv < m1, pv, -1.0))

            m2 = lax.fori_loop(0, _E, _m2, jnp.full((_L,), -1.0, jnp.float32))

            wv = wb[sl]

            def _route(i, carry):
                pi = pbuf[i, sl]

                def _gt(j, s):
                    pj = pbuf[j, sl]
                    return s + jnp.where(pj > pi, pj, 0.0)

                s = lax.fori_loop(0, _E, _gt, zeros)
                keep = high | (pi >= m2) | ((s + pi) <= _TOP_P)
                rw = jnp.where(keep, pi, 0.0)
                acc[i, :] += rw * wv
                acc[_E + i, :] += pi * wv
                return carry

            lax.fori_loop(0, _E, _route, 0)
            return (S, T, D + wv)

        S, T, D = lax.fori_loop(0, _NG, _group, (S, T, D))

    acc[2 * _E, :] = S
    acc[2 * _E + 1, :] = T
    acc[2 * _E + 2, :] = D
    acc[2 * _E + 3, :] = zeros
    pltpu.sync_copy(acc, out_hbm.at[wid])


def _combine_body(p_ref, out_ref):
    p = p_ref[...]
    pm = jnp.sum(p, axis=0)
    a = jnp.sum(pm[0:_E, :], axis=1, keepdims=True)
    b = jnp.sum(pm[_E:2 * _E, :], axis=1, keepdims=True)
    s = jnp.sum(pm[2 * _E:2 * _E + 1, :], axis=1, keepdims=True)
    t = jnp.sum(pm[2 * _E + 1:2 * _E + 2, :], axis=1, keepdims=True)
    d = jnp.sum(pm[2 * _E + 2:2 * _E + 3, :], axis=1, keepdims=True)
    ent = (1.0 - t / (s ** _Q)) / (_Q - 1.0)
    lb = _E * jnp.sum(a * b, axis=0, keepdims=True) / (d * d)
    out_ref[...] = 0.001 * ent + 0.001 * lb


def kernel(gate_logits, attention_mask):
    n, e = gate_logits.shape
    bsz, seq = attention_mask.shape
    layers = n // (bsz * seq)

    x_r = gate_logits.reshape(_NW, _NCH, _CH, e).transpose(0, 1, 3, 2)
    w_r = jnp.broadcast_to(
        attention_mask.reshape(-1)[None, :], (layers, bsz * seq)
    ).reshape(_NW, _NCH, _CH).astype(jnp.float32)

    mesh = plsc.VectorSubcoreMesh(
        core_axis_name="c", subcore_axis_name="s",
        num_cores=_NC, num_subcores=_NS)
    partials = pl.kernel(
        _sc_body,
        out_type=jax.ShapeDtypeStruct((_NW, _PR, _L), jnp.float32),
        mesh=mesh,
        scratch_types=[
            pltpu.VMEM((_E, _CH), jnp.float32),
            pltpu.VMEM((_CH,), jnp.float32),
            pltpu.VMEM((_E, _CH), jnp.float32),
            pltpu.VMEM((_PR, _L), jnp.float32),
        ],
    )(x_r, w_r)

    loss = pl.pallas_call(
        _combine_body,
        out_shape=jax.ShapeDtypeStruct((1, 1), jnp.float32),
    )(partials)
    return loss.reshape(())


# speedup vs baseline: 4.7179x; 4.7179x over previous
import functools

import jax
import jax.numpy as jnp
from jax import lax
from jax.experimental import pallas as pl
from jax.experimental.pallas import tpu as pltpu
from jax.experimental.pallas import tpu_sc as plsc

_E = 64
_Q = 1.2
_ENT_TH = 2.5
_TOP_P = 0.75

_NC, _NS, _L = 2, 16, 16
_NW = _NC * _NS
_N = 65536
_TPW = _N // _NW
_CH = 512
_NCH = _TPW // _CH
_NG = _CH // _L
_PR = 2 * _E + 4


def _sc_body(x_hbm, w_hbm, out_hbm, xb, wb, pbuf, acc):
    wid = lax.axis_index("s") * _NC + lax.axis_index("c")
    zeros = jnp.zeros((_L,), jnp.float32)

    def _zinit(j, carry):
        acc[j, :] = zeros
        return carry

    lax.fori_loop(0, _PR, _zinit, 0)

    S, T, D = zeros, zeros, zeros
    for c in range(_NCH):
        pltpu.sync_copy(x_hbm.at[wid, c], xb)
        pltpu.sync_copy(w_hbm.at[wid, c], wb)

        def _group(g, std):
            S, T, D = std
            sl = pl.ds(g * _L, _L)
            ninf = jnp.full((_L,), -1e30, jnp.float32)

            def _p1(blk, carry):
                m, m2x = carry
                for k in range(8):
                    x = xb[blk * 8 + k, sl]
                    m2x = jnp.maximum(m2x, jnp.minimum(m, x))
                    m = jnp.maximum(m, x)
                return m, m2x

            m, m2x = lax.fori_loop(0, 8, _p1, (ninf, ninf))

            def _p2(blk, z):
                for k in range(8):
                    j = blk * 8 + k
                    e = jnp.exp(xb[j, sl] - m)
                    pbuf[j, sl] = e
                    z = z + e
                return z

            z = lax.fori_loop(0, 8, _p2, zeros)
            rz = 1.0 / z

            y0 = jnp.where(z >= 20.0855, 3.5,
                           jnp.where(z >= 7.3891, 2.5,
                                     jnp.where(z >= 2.7183, 1.5, 0.5)))

            def _newton(_, y):
                return y - 1.0 + z * jnp.exp(-y)

            lnz = lax.fori_loop(0, 5, _newton, y0)
            k12 = _Q * (m + lnz)
            e2 = jnp.exp(m2x - m)

            def _p4(blk, carry):
                S, pqs = carry
                for k in range(8):
                    j = blk * 8 + k
                    ev = pbuf[j, sl]
                    S = S + jnp.maximum(ev * rz, 1e-5)
                    pq = jnp.exp(_Q * xb[j, sl] - k12)
                    pqs = pqs + jnp.maximum(pq, 1e-6)
                return S, pqs

            S, pqs = lax.fori_loop(0, 8, _p4, (S, zeros))
            T = T + pqs
            high = ((1.0 - pqs) / (_Q - 1.0)) >= _ENT_TH

            thr = _TOP_P * z

            def _bis(_, lh):
                lo, hi = lh
                u = 0.5 * (lo + hi)

                def _gsum(blk, gs):
                    for k in range(8):
                        ev = pbuf[blk * 8 + k, sl]
                        gs = gs + jnp.where(ev >= u, ev, 0.0)
                    return gs

                gs = lax.fori_loop(0, 8, _gsum, zeros)
                ok = gs <= thr
                return jnp.where(ok, lo, u), jnp.where(ok, u, hi)

            _, hi = lax.fori_loop(
                0, 18, _bis, (zeros, jnp.full((_L,), 2.0, jnp.float32)))

            wv = wb[sl]
            rzw = rz * wv

            def _p6(blk, carry):
                for k in range(8):
                    j = blk * 8 + k
                    ev = pbuf[j, sl]
                    keep = high | (ev >= e2) | (ev >= hi)
                    rwv = jnp.where(keep, ev, 0.0)
                    plsc.addupdate(acc.at[j, :], rwv * rzw)
                    plsc.addupdate(acc.at[_E + j, :], ev * rzw)
                return carry

            lax.fori_loop(0, 8, _p6, 0)
            return (S, T, D + wv)

        S, T, D = lax.fori_loop(0, _NG, _group, (S, T, D))

    acc[2 * _E, :] = S
    acc[2 * _E + 1, :] = T
    acc[2 * _E + 2, :] = D
    acc[2 * _E + 3, :] = zeros
    pltpu.sync_copy(acc, out_hbm.at[wid])


def _combine_body(p_ref, out_ref):
    p = p_ref[...]
    pm = jnp.sum(p, axis=0)
    a = jnp.sum(pm[0:_E, :], axis=1, keepdims=True)
    b = jnp.sum(pm[_E:2 * _E, :], axis=1, keepdims=True)
    s = jnp.sum(pm[2 * _E:2 * _E + 1, :], axis=1, keepdims=True)
    t = jnp.sum(pm[2 * _E + 1:2 * _E + 2, :], axis=1, keepdims=True)
    d = jnp.sum(pm[2 * _E + 2:2 * _E + 3, :], axis=1, keepdims=True)
    ent = (1.0 - t / (s ** _Q)) / (_Q - 1.0)
    lb = _E * jnp.sum(a * b, axis=0, keepdims=True) / (d * d)
    out_ref[...] = 0.001 * ent + 0.001 * lb


def kernel(gate_logits, attention_mask):
    n, e = gate_logits.shape
    bsz, seq = attention_mask.shape
    layers = n // (bsz * seq)

    x_r = gate_logits.reshape(_NW, _NCH, _CH, e).transpose(0, 1, 3, 2)
    w_r = jnp.broadcast_to(
        attention_mask.reshape(-1)[None, :], (layers, bsz * seq)
    ).reshape(_NW, _NCH, _CH).astype(jnp.float32)

    mesh = plsc.VectorSubcoreMesh(
        core_axis_name="c", subcore_axis_name="s",
        num_cores=_NC, num_subcores=_NS)
    partials = pl.kernel(
        _sc_body,
        out_type=jax.ShapeDtypeStruct((_NW, _PR, _L), jnp.float32),
        mesh=mesh,
        scratch_types=[
            pltpu.VMEM((_E, _CH), jnp.float32),
            pltpu.VMEM((_CH,), jnp.float32),
            pltpu.VMEM((_E, _CH), jnp.float32),
            pltpu.VMEM((_PR, _L), jnp.float32),
        ],
    )(x_r, w_r)

    loss = pl.pallas_call(
        _combine_body,
        out_shape=jax.ShapeDtypeStruct((1, 1), jnp.float32),
    )(partials)
    return loss.reshape(())
